# Initial kernel scaffold; baseline (speedup 1.0000x reference)
import jax
import jax.numpy as jnp
from jax import lax
from jax.experimental import pallas as pl
from jax.experimental.pallas import tpu as pltpu
from jax.experimental.pallas import tpu_sc as plsc

N_NODES = 10000
IN_DIM = 128
HIDDEN = 64
HEADS = 4
OUT_PER_HEAD = 16
N_EDGES = 320000

NW = 32
BATCH = 128
PER_TILE = 10368
NB = PER_TILE // BATCH
EDGES_PAD = PER_TILE * NW
TSW = 72
TDW = 16
ACC_ROWS = 10016
ROWS_PER_TILE = ACC_ROWS // 16


def _dense_tables_kernel(x_ref, w_ref, m_src_ref, m_dst_ref, ts_ref, td_ref):
    h = jnp.dot(x_ref[...], w_ref[...], preferred_element_type=jnp.float32)
    ts_ref[...] = jnp.dot(h, m_src_ref[...], preferred_element_type=jnp.float32)
    td_ref[...] = jnp.dot(h, m_dst_ref[...], preferred_element_type=jnp.float32)


def _dense_tables(x, w, m_src, m_dst):
    n, k = x.shape
    blk = 2000
    grid = n // blk
    return pl.pallas_call(
        _dense_tables_kernel,
        grid=(grid,),
        in_specs=[
            pl.BlockSpec((blk, k), lambda i: (i, 0)),
            pl.BlockSpec((k, HIDDEN), lambda i: (0, 0)),
            pl.BlockSpec((HIDDEN, TSW), lambda i: (0, 0)),
            pl.BlockSpec((HIDDEN, TDW), lambda i: (0, 0)),
        ],
        out_specs=[
            pl.BlockSpec((blk, TSW), lambda i: (i, 0)),
            pl.BlockSpec((blk, TDW), lambda i: (i, 0)),
        ],
        out_shape=[
            jax.ShapeDtypeStruct((n, TSW), jnp.float32),
            jax.ShapeDtypeStruct((n, TDW), jnp.float32),
        ],
    )(x, w, m_src, m_dst)


def _edge_pass_body(tsrc_hbm, tdst_hbm, src_hbm, dst_hbm, zeros_hbm, out_hbm,
                    acc, sidx, didx, srows, drows, sem1, sem2):
    c = lax.axis_index("c")
    s = lax.axis_index("s")
    w = c * 16 + s

    r0 = s * ROWS_PER_TILE
    pltpu.sync_copy(zeros_hbm.at[pl.ds(r0, ROWS_PER_TILE)],
                    acc.at[pl.ds(r0, ROWS_PER_TILE)])
    plsc.subcore_barrier()

    lanes = lax.broadcasted_iota(jnp.int32, (16,), 0)

    def batch_body(i, carry):
        base = w * PER_TILE + i * BATCH
        pltpu.sync_copy(src_hbm.at[pl.ds(base, BATCH)], sidx)
        pltpu.sync_copy(dst_hbm.at[pl.ds(base, BATCH)], didx)
        cp1 = pltpu.async_copy(tsrc_hbm.at[sidx], srows, sem1)
        cp2 = pltpu.async_copy(tdst_hbm.at[didx], drows, sem2)
        cp1.wait()
        cp2.wait()

        for hh in range(HEADS):
            col_s = jnp.full((16,), HIDDEN + hh, jnp.int32)
            col_d = jnp.full((16,), hh, jnp.int32)
            for ck in range(BATCH // 16):
                rows = lanes + ck * 16
                av = plsc.load_gather(srows, [rows, col_s])
                dv = plsc.load_gather(drows, [rows, col_d])
                t = av + dv
                t = jnp.maximum(t, 0.2 * t)
                plsc.store_scatter(srows, [rows, col_s], jnp.exp(t))

        def scale_body(e, carry2):
            for hh in range(HEADS):
                sc = srows[e, HIDDEN + hh]
                v = srows[e, pl.ds(hh * OUT_PER_HEAD, 16)]
                srows[e, pl.ds(hh * OUT_PER_HEAD, 16)] = v * sc
            return carry2

        lax.fori_loop(0, BATCH, scale_body, 0, unroll=2)

        pltpu.sync_copy(srows, acc.at[didx], add=True)
        return carry

    lax.fori_loop(0, NB, batch_body, 0)

    plsc.subcore_barrier()
    pltpu.sync_copy(acc.at[pl.ds(r0, ROWS_PER_TILE)],
                    out_hbm.at[c, pl.ds(r0, ROWS_PER_TILE)])


def _edge_pass(tsrc, tdst_pad, src_all, dst_all, zeros_acc):
    mesh = plsc.VectorSubcoreMesh(core_axis_name="c", subcore_axis_name="s")
    return pl.kernel(
        _edge_pass_body,
        out_type=jax.ShapeDtypeStruct((2, ACC_ROWS, TSW), jnp.float32),
        mesh=mesh,
        scratch_types=[
            pltpu.VMEM_SHARED((ACC_ROWS, TSW), jnp.float32),
            pltpu.VMEM((BATCH,), jnp.int32),
            pltpu.VMEM((BATCH,), jnp.int32),
            pltpu.VMEM((BATCH, TSW), jnp.float32),
            pltpu.VMEM((BATCH, TDW), jnp.float32),
            pltpu.SemaphoreType.DMA,
            pltpu.SemaphoreType.DMA,
        ],
    )(tsrc, tdst_pad, src_all, dst_all, zeros_acc)


def _finish(acc, bias, gamma, beta, e4):
    a = acc[0]
    b = acc[1]
    y = a[:N_NODES, :HIDDEN] + b[:N_NODES, :HIDDEN]
    den = a[:N_NODES, HIDDEN:HIDDEN + HEADS] + b[:N_NODES, HIDDEN:HIDDEN + HEADS]
    denb = jnp.dot(den, e4, preferred_element_type=jnp.float32)
    o = y / (denb + 1e-16) + bias
    mu = jnp.mean(o, axis=0, keepdims=True)
    var = jnp.mean((o - mu) * (o - mu), axis=0, keepdims=True)
    z = gamma * (o - mu) * lax.rsqrt(var + 1e-5) + beta
    return jnp.maximum(z, 0.0)


def _epilogue_dense_kernel(acc_ref, b_ref, g_ref, be_ref, e4_ref,
                           w2_ref, m_src_ref, m_dst_ref, ts_ref, td_ref):
    z = _finish(acc_ref[...], b_ref[...], g_ref[...], be_ref[...], e4_ref[...])
    h2 = jnp.dot(z, w2_ref[...], preferred_element_type=jnp.float32)
    ts_ref[...] = jnp.dot(h2, m_src_ref[...], preferred_element_type=jnp.float32)
    td_ref[...] = jnp.dot(h2, m_dst_ref[...], preferred_element_type=jnp.float32)


def _epilogue_final_kernel(acc_ref, b_ref, g_ref, be_ref, e4_ref, out_ref):
    out_ref[...] = _finish(acc_ref[...], b_ref[...], g_ref[...], be_ref[...],
                           e4_ref[...])


def _epilogue_dense(acc, bias, gamma, beta, e4, w2, m_src, m_dst):
    return pl.pallas_call(
        _epilogue_dense_kernel,
        out_shape=[
            jax.ShapeDtypeStruct((N_NODES, TSW), jnp.float32),
            jax.ShapeDtypeStruct((N_NODES, TDW), jnp.float32),
        ],
    )(acc, bias, gamma, beta, e4, w2, m_src, m_dst)


def _epilogue_final(acc, bias, gamma, beta, e4):
    return pl.pallas_call(
        _epilogue_final_kernel,
        out_shape=jax.ShapeDtypeStruct((N_NODES, HIDDEN), jnp.float32),
    )(acc, bias, gamma, beta, e4)


def _att_mat(att):
    rows = jnp.arange(HIDDEN)
    cols = jnp.repeat(jnp.arange(HEADS), OUT_PER_HEAD)
    return jnp.zeros((HIDDEN, HEADS), jnp.float32).at[rows, cols].set(
        att.reshape(HIDDEN))


def kernel(x, edge_index, W1, att_src1, att_dst1, b1, g1, be1,
           W2, att_src2, att_dst2, b2, g2, be2):
    n = x.shape[0]
    i32 = jnp.int32

    loop = jnp.arange(n, dtype=i32)
    pad = EDGES_PAD - (N_EDGES + n)
    src_all = jnp.concatenate([
        edge_index[0].astype(i32), loop, jnp.zeros((pad,), i32)])
    dst_all = jnp.concatenate([
        edge_index[1].astype(i32), loop, jnp.full((pad,), n, i32)])

    eye = jnp.eye(HIDDEN, dtype=jnp.float32)
    zpad4 = jnp.zeros((HIDDEN, 4), jnp.float32)
    m_src1 = jnp.concatenate([eye, _att_mat(att_src1), zpad4], axis=1)
    m_dst1 = jnp.concatenate(
        [_att_mat(att_dst1), jnp.zeros((HIDDEN, TDW - HEADS), jnp.float32)],
        axis=1)
    m_src2 = jnp.concatenate([eye, _att_mat(att_src2), zpad4], axis=1)
    m_dst2 = jnp.concatenate(
        [_att_mat(att_dst2), jnp.zeros((HIDDEN, TDW - HEADS), jnp.float32)],
        axis=1)
    e4 = jnp.repeat(jnp.eye(HEADS, dtype=jnp.float32), OUT_PER_HEAD, axis=1)
    zeros_acc = jnp.zeros((ACC_ROWS, TSW), jnp.float32)
    td_zpad = jnp.zeros((8, TDW), jnp.float32)

    b1r = b1.reshape(1, HIDDEN)
    g1r = g1.reshape(1, HIDDEN)
    be1r = be1.reshape(1, HIDDEN)
    b2r = b2.reshape(1, HIDDEN)
    g2r = g2.reshape(1, HIDDEN)
    be2r = be2.reshape(1, HIDDEN)

    ts1, td1 = _dense_tables(x, W1, m_src1, m_dst1)
    td1p = jnp.concatenate([td1, td_zpad], axis=0)
    acc1 = _edge_pass(ts1, td1p, src_all, dst_all, zeros_acc)
    ts2, td2 = _epilogue_dense(acc1, b1r, g1r, be1r, e4, W2, m_src2, m_dst2)

    td2p = jnp.concatenate([td2, td_zpad], axis=0)
    acc2 = _edge_pass(ts2, td2p, src_all, dst_all, zeros_acc)
    return _epilogue_final(acc2, b2r, g2r, be2r, e4)


# speedup vs baseline: 60.5990x; 60.5990x over previous
import jax
import jax.numpy as jnp
from jax import lax
from jax.experimental import pallas as pl
from jax.experimental.pallas import tpu as pltpu
from jax.experimental.pallas import tpu_sc as plsc

N_NODES = 10000
HIDDEN = 64
HEADS = 4
OUT_PER_HEAD = 16
N_EDGES = 320000

NW = 32
BATCH = 128
PER_TILE = 10368
NB = PER_TILE // BATCH
EDGES_PAD = PER_TILE * NW
TSW = 80
TDW = 16
ACC_ROWS = 10112
ROWS_PER_TILE = ACC_ROWS // 16


def _dense_tables_kernel(x_ref, w_ref, m_src_ref, m_dst_ref, ts_ref, td_ref):
    h = jnp.dot(x_ref[...], w_ref[...], preferred_element_type=jnp.float32)
    ts_ref[...] = jnp.dot(h, m_src_ref[...], preferred_element_type=jnp.float32)
    td_ref[...] = jnp.dot(h, m_dst_ref[...], preferred_element_type=jnp.float32)


def _dense_tables(x, w, m_src, m_dst):
    n, k = x.shape
    blk = 2000
    return pl.pallas_call(
        _dense_tables_kernel,
        grid=(n // blk,),
        in_specs=[
            pl.BlockSpec((blk, k), lambda i: (i, 0)),
            pl.BlockSpec((k, HIDDEN), lambda i: (0, 0)),
            pl.BlockSpec((HIDDEN, TSW), lambda i: (0, 0)),
            pl.BlockSpec((HIDDEN, TDW), lambda i: (0, 0)),
        ],
        out_specs=[
            pl.BlockSpec((blk, TSW), lambda i: (i, 0)),
            pl.BlockSpec((blk, TDW), lambda i: (i, 0)),
        ],
        out_shape=[
            jax.ShapeDtypeStruct((n, TSW), jnp.float32),
            jax.ShapeDtypeStruct((n, TDW), jnp.float32),
        ],
    )(x, w, m_src, m_dst)


def _edge_pass_body(tsrc_hbm, tdst_hbm, src_hbm, dst_hbm, zeros_hbm, out_hbm,
                    acc, sidx, didx, srows, drows, sem1, sem2):
    c = lax.axis_index("c")
    s = lax.axis_index("s")
    w = c * 16 + s

    r0 = s * ROWS_PER_TILE
    pltpu.sync_copy(zeros_hbm.at[pl.ds(r0, ROWS_PER_TILE)],
                    acc.at[pl.ds(r0, ROWS_PER_TILE)])
    plsc.subcore_barrier()

    def batch_body(i, carry):
        base = w * PER_TILE + i * BATCH
        pltpu.sync_copy(src_hbm.at[pl.ds(base, BATCH)], sidx)
        pltpu.sync_copy(dst_hbm.at[pl.ds(base, BATCH)], didx)
        cp1 = pltpu.async_copy(tsrc_hbm.at[sidx], srows, sem1)
        cp2 = pltpu.async_copy(tdst_hbm.at[didx], drows, sem2)
        cp1.wait()
        cp2.wait()

        def scale_body(e, carry2):
            av = srows[e, pl.ds(HIDDEN - 8, 16)]
            dv = drows[e, pl.ds(0, 16)]
            t = av + dv
            t = jnp.maximum(t, 0.2 * t)
            e16 = jnp.exp(t)
            for hh in range(HEADS):
                sc = e16[8 + hh]
                v = srows[e, pl.ds(hh * OUT_PER_HEAD, 16)]
                srows[e, pl.ds(hh * OUT_PER_HEAD, 16)] = v * sc
            srows[e, pl.ds(HIDDEN, 16)] = e16
            return carry2

        lax.fori_loop(0, BATCH, scale_body, 0, unroll=2)

        pltpu.sync_copy(srows, acc.at[didx], add=True)
        return carry

    lax.fori_loop(0, NB, batch_body, 0)

    plsc.subcore_barrier()
    pltpu.sync_copy(acc.at[pl.ds(r0, ROWS_PER_TILE)],
                    out_hbm.at[c, pl.ds(r0, ROWS_PER_TILE)])


def _edge_pass(tsrc, tdst_pad, src_all, dst_all, zeros_acc):
    mesh = plsc.VectorSubcoreMesh(core_axis_name="c", subcore_axis_name="s")
    return pl.kernel(
        _edge_pass_body,
        out_type=jax.ShapeDtypeStruct((2, ACC_ROWS, TSW), jnp.float32),
        mesh=mesh,
        scratch_types=[
            pltpu.VMEM_SHARED((ACC_ROWS, TSW), jnp.float32),
            pltpu.VMEM((BATCH,), jnp.int32),
            pltpu.VMEM((BATCH,), jnp.int32),
            pltpu.VMEM((BATCH, TSW), jnp.float32),
            pltpu.VMEM((BATCH, TDW), jnp.float32),
            pltpu.SemaphoreType.DMA,
            pltpu.SemaphoreType.DMA,
        ],
        compiler_params=pltpu.CompilerParams(use_tc_tiling_on_sc=False),
    )(tsrc, tdst_pad, src_all, dst_all, zeros_acc)


def _finish(acc, bias, gamma, beta, e4):
    a = acc[0]
    b = acc[1]
    y = a[:N_NODES, :HIDDEN] + b[:N_NODES, :HIDDEN]
    den = a[:N_NODES, 72:76] + b[:N_NODES, 72:76]
    denb = jnp.dot(den, e4, preferred_element_type=jnp.float32)
    o = y / (denb + 1e-16) + bias
    mu = jnp.mean(o, axis=0, keepdims=True)
    var = jnp.mean((o - mu) * (o - mu), axis=0, keepdims=True)
    z = gamma * (o - mu) * lax.rsqrt(var + 1e-5) + beta
    return jnp.maximum(z, 0.0)


def _epilogue_dense_kernel(acc_ref, b_ref, g_ref, be_ref, e4_ref,
                           w2_ref, m_src_ref, m_dst_ref, ts_ref, td_ref):
    z = _finish(acc_ref[...], b_ref[...], g_ref[...], be_ref[...], e4_ref[...])
    h2 = jnp.dot(z, w2_ref[...], preferred_element_type=jnp.float32)
    ts_ref[...] = jnp.dot(h2, m_src_ref[...], preferred_element_type=jnp.float32)
    td_ref[...] = jnp.dot(h2, m_dst_ref[...], preferred_element_type=jnp.float32)


def _epilogue_final_kernel(acc_ref, b_ref, g_ref, be_ref, e4_ref, out_ref):
    out_ref[...] = _finish(acc_ref[...], b_ref[...], g_ref[...], be_ref[...],
                           e4_ref[...])


def _epilogue_dense(acc, bias, gamma, beta, e4, w2, m_src, m_dst):
    return pl.pallas_call(
        _epilogue_dense_kernel,
        out_shape=[
            jax.ShapeDtypeStruct((N_NODES, TSW), jnp.float32),
            jax.ShapeDtypeStruct((N_NODES, TDW), jnp.float32),
        ],
    )(acc, bias, gamma, beta, e4, w2, m_src, m_dst)


def _epilogue_final(acc, bias, gamma, beta, e4):
    return pl.pallas_call(
        _epilogue_final_kernel,
        out_shape=jax.ShapeDtypeStruct((N_NODES, HIDDEN), jnp.float32),
    )(acc, bias, gamma, beta, e4)


def _att_mat(att):
    rows = jnp.arange(HIDDEN)
    cols = jnp.repeat(jnp.arange(HEADS), OUT_PER_HEAD)
    return jnp.zeros((HIDDEN, HEADS), jnp.float32).at[rows, cols].set(
        att.reshape(HIDDEN))


def kernel(x, edge_index, W1, att_src1, att_dst1, b1, g1, be1,
           W2, att_src2, att_dst2, b2, g2, be2):
    n = x.shape[0]
    i32 = jnp.int32

    loop = jnp.arange(n, dtype=i32)
    pad = EDGES_PAD - (N_EDGES + n)
    src_all = jnp.concatenate([
        edge_index[0].astype(i32), loop, jnp.zeros((pad,), i32)])
    dst_all = jnp.concatenate([
        edge_index[1].astype(i32), loop, jnp.full((pad,), n, i32)])

    eye = jnp.eye(HIDDEN, dtype=jnp.float32)
    zpad12 = jnp.zeros((HIDDEN, 12), jnp.float32)
    zpad8 = jnp.zeros((HIDDEN, 8), jnp.float32)
    zpad4 = jnp.zeros((HIDDEN, 4), jnp.float32)
    m_src1 = jnp.concatenate([eye, _att_mat(att_src1), zpad12], axis=1)
    m_dst1 = jnp.concatenate([zpad8, _att_mat(att_dst1), zpad4], axis=1)
    m_src2 = jnp.concatenate([eye, _att_mat(att_src2), zpad12], axis=1)
    m_dst2 = jnp.concatenate([zpad8, _att_mat(att_dst2), zpad4], axis=1)
    e4 = jnp.repeat(jnp.eye(HEADS, dtype=jnp.float32), OUT_PER_HEAD, axis=1)
    zeros_acc = jnp.zeros((ACC_ROWS, TSW), jnp.float32)
    td_zpad = jnp.zeros((8, TDW), jnp.float32)

    b1r = b1.reshape(1, HIDDEN)
    g1r = g1.reshape(1, HIDDEN)
    be1r = be1.reshape(1, HIDDEN)
    b2r = b2.reshape(1, HIDDEN)
    g2r = g2.reshape(1, HIDDEN)
    be2r = be2.reshape(1, HIDDEN)

    ts1, td1 = _dense_tables(x, W1, m_src1, m_dst1)
    td1p = jnp.concatenate([td1, td_zpad], axis=0)
    acc1 = _edge_pass(ts1, td1p, src_all, dst_all, zeros_acc)
    ts2, td2 = _epilogue_dense(acc1, b1r, g1r, be1r, e4, W2, m_src2, m_dst2)

    td2p = jnp.concatenate([td2, td_zpad], axis=0)
    acc2 = _edge_pass(ts2, td2p, src_all, dst_all, zeros_acc)
    return _epilogue_final(acc2, b2r, g2r, be2r, e4)
